# Initial kernel scaffold; baseline (speedup 1.0000x reference)
import jax
import jax.numpy as jnp
from jax.experimental import pallas as pl


def kernel(kernel, inputs):
    raise NotImplementedError("write your pallas kernel here")


# speedup vs baseline: 1.1041x; 1.1041x over previous
import functools

import jax
import jax.numpy as jnp
from jax import lax
from jax.experimental import pallas as pl
from jax.experimental.pallas import tpu as pltpu
from jax.experimental.pallas import tpu_sc as plsc

_START = 1024
_ROWS = 1002048
_D = 64
_B = 4096 * 50
_NC, _NS = 2, 16
_NW = _NC * _NS
_CHUNK = 128
_PER_W = _B // _NW
_NCHUNK = _PER_W // _CHUNK
_L = 16


def _emb_kernel(table_hbm, idx_hbm, out_hbm, idx_v, buf_v, sem):
    wid = lax.axis_index("s") * _NC + lax.axis_index("c")
    row0 = wid * _PER_W

    pltpu.sync_copy(idx_hbm.at[wid], idx_v)

    def add_body(i, carry):
        for k in range(_CHUNK // _L):
            sl = pl.ds(k * _L, _L)
            idx_v[i, sl] = idx_v[i, sl] + _START
        return carry

    lax.fori_loop(0, _NCHUNK, add_body, 0)

    def gather_body(j, carry):
        pltpu.async_copy(table_hbm.at[idx_v.at[j]], buf_v, sem).wait()
        pltpu.sync_copy(buf_v, out_hbm.at[pl.ds(row0 + j * _CHUNK, _CHUNK)])
        return carry

    lax.fori_loop(0, _NCHUNK, gather_body, 0)


@jax.jit
def _lookup(table, idx3d):
    mesh = plsc.VectorSubcoreMesh(core_axis_name="c", subcore_axis_name="s")
    f = functools.partial(
        pl.kernel,
        mesh=mesh,
        compiler_params=pltpu.CompilerParams(use_tc_tiling_on_sc=False),
        out_type=jax.ShapeDtypeStruct((_B, _D), jnp.float32),
        scratch_types=[
            pltpu.VMEM((_NCHUNK, _CHUNK), jnp.int32),
            pltpu.VMEM((_CHUNK, _D), jnp.float32),
            pltpu.SemaphoreType.DMA,
        ],
    )(_emb_kernel)
    return f(table, idx3d)


def kernel(kernel, inputs):
    idx3d = inputs.reshape(_NW, _NCHUNK, _CHUNK)
    out = _lookup(kernel, idx3d)
    return out.reshape(inputs.shape[0], inputs.shape[1], _D)
